# baseline (speedup 1.0000x reference)
import functools

import jax
import jax.numpy as jnp
from jax import lax
from jax.experimental import pallas as pl
from jax.experimental.pallas import tpu as pltpu
from jax.experimental.pallas import tpu_sc as plsc

N = 10000
E = 320000
FIN = 128
HID = 128
NCLS = 64

NC = 2
NS = 16
NW = NC * NS

NPAD = 10240
EPAD = 327680
CHUNKS = 80
K = 128
RPT = NPAD // NS

_MESH = plsc.VectorSubcoreMesh(
    core_axis_name="c", subcore_axis_name="s", num_cores=NC, num_subcores=NS)

_SC_PARAMS = pltpu.CompilerParams(use_tc_tiling_on_sc=False)


def _worker_id():
  return lax.axis_index("c") * NS + lax.axis_index("s")


@functools.partial(
    pl.kernel,
    out_type=jax.ShapeDtypeStruct((NC, NPAD, 16), jnp.float32),
    mesh=_MESH,
    compiler_params=_SC_PARAMS,
    scratch_types=[
        pltpu.VMEM((CHUNKS, K), jnp.int32),
        pltpu.VMEM((K, 16), jnp.float32),
        pltpu.VMEM((RPT, 16), jnp.float32),
        pltpu.VMEM_SHARED((NPAD, 16), jnp.float32),
    ],
)
def _sc_deg(dst3, ones16, zeros16, out, dstv, onesv, zbuf, accum):
  cid = lax.axis_index("c")
  sid = lax.axis_index("s")
  wid = _worker_id()
  pltpu.sync_copy(dst3.at[wid], dstv)
  pltpu.sync_copy(ones16, onesv)
  pltpu.sync_copy(zeros16, zbuf)
  pltpu.sync_copy(zbuf, accum.at[pl.ds(sid * RPT, RPT)])
  plsc.subcore_barrier()

  def body(j, carry):
    pltpu.sync_copy(onesv, accum.at[dstv.at[j]], add=True)
    return carry

  lax.fori_loop(0, CHUNKS, body, 0)
  plsc.subcore_barrier()
  pltpu.sync_copy(accum.at[pl.ds(sid * RPT, RPT)], zbuf)
  pltpu.sync_copy(zbuf, out.at[cid, pl.ds(sid * RPT, RPT)])


NBUF = 2
NIDX = 2 * NBUF


def _make_sc_scatter(d, nphases):
  scratch = [
      [pltpu.VMEM((2, K), jnp.int32)] * NIDX,
      [pltpu.VMEM((K, d), jnp.float32)] * NBUF,
      pltpu.VMEM_SHARED((NPAD, d), jnp.float32),
      pltpu.VMEM_SHARED((NPAD, d), jnp.float32),
      [pltpu.SemaphoreType.DMA] * NIDX,
      [pltpu.SemaphoreType.DMA] * NBUF,
      [pltpu.SemaphoreType.DMA] * NBUF,
  ]

  @functools.partial(
      pl.kernel,
      out_type=jax.ShapeDtypeStruct((nphases, NC, NPAD, d), jnp.float32),
      mesh=_MESH,
      compiler_params=_SC_PARAMS,
      scratch_types=scratch,
  )
  def sc_scatter(*args):
    tables = args[:nphases]
    edges4, zrows, out = args[nphases:nphases + 3]
    ibuf, rows, accum, g, isem, gsem, ssem = args[nphases + 3:]
    cid = lax.axis_index("c")
    sid = lax.axis_index("s")
    wid = _worker_id()
    nzero = RPT // K

    for p in range(nphases):
      for q in range(nzero):
        r0 = sid * RPT + q * K
        pltpu.sync_copy(tables[p].at[pl.ds(r0, K)], rows[0])
        pltpu.sync_copy(rows[0], g.at[pl.ds(r0, K)])
      pltpu.sync_copy(zrows, rows[0])
      for q in range(nzero):
        pltpu.sync_copy(rows[0], accum.at[pl.ds(sid * RPT + q * K, K)])
      plsc.subcore_barrier()

      for u in range(NIDX):
        pltpu.async_copy(edges4.at[wid, u], ibuf[u], isem[u])
      for b in range(NBUF):
        pltpu.make_async_copy(edges4.at[wid, 0], ibuf[b], isem[b]).wait()
        pltpu.async_copy(g.at[ibuf[b].at[0]], rows[b], gsem[b])

      def group(gi, carry):
        base = gi * NIDX
        for u in range(NIDX):
          j = base + u
          b = u % NBUF
          pltpu.make_async_copy(zrows, rows[b], gsem[b]).wait()
          pltpu.async_copy(rows[b], accum.at[ibuf[u].at[1]], ssem[b],
                           add=True)
          pltpu.make_async_copy(zrows, rows[b], ssem[b]).wait()
          nj = j + NBUF
          nslot = (u + NBUF) % NIDX

          @pl.when(nj < CHUNKS)
          def _():
            pltpu.make_async_copy(edges4.at[wid, 0], ibuf[nslot],
                                  isem[nslot]).wait()
            pltpu.async_copy(g.at[ibuf[nslot].at[0]], rows[b], gsem[b])

          pj = j + NIDX

          @pl.when(pj < CHUNKS)
          def _():
            pltpu.async_copy(edges4.at[wid, pj], ibuf[u], isem[u])

        return carry

      lax.fori_loop(0, CHUNKS // NIDX, group, 0)
      plsc.subcore_barrier()
      for q in range(nzero):
        r0 = sid * RPT + q * K
        pltpu.sync_copy(accum.at[pl.ds(r0, K)], rows[0])
        pltpu.sync_copy(rows[0], out.at[p, cid, pl.ds(r0, K)])

  return sc_scatter


_sc_scatter_l1 = _make_sc_scatter(NCLS, nphases=2)
_sc_scatter_l2 = _make_sc_scatter(NCLS, nphases=1)


R = 1024


def _tc1_body(x, w1, degp, rates_p, we1_p, be1_p, we2_p, be2_p, w2b,
              g1a_out, g1b_out, c_out):
  deg = degp[0, :, 0] + degp[1, :, 0] + 1.0
  dinv = lax.rsqrt(deg)
  h = jnp.dot(x[...], w1[...], preferred_element_type=jnp.float32)
  g1 = h * dinv[:, None]
  g1a_out[...] = g1[:, :NCLS]
  g1b_out[...] = g1[:, NCLS:]
  he = jnp.maximum(
      jnp.dot(rates_p[...], we1_p[...], preferred_element_type=jnp.float32)
      + be1_p[...], 0.0)
  rep = jnp.dot(he, we2_p[...], preferred_element_type=jnp.float32) + be2_p[...]
  c_out[...] = jnp.dot(rep, w2b[...], preferred_element_type=jnp.float32)


def _tc2_body(g1a, g1b, degp, s1, b1_p, w2t, c, g2_out):
  deg = degp[0, :, 0] + degp[1, :, 0] + 1.0
  dinv = lax.rsqrt(deg)
  acc = jnp.concatenate(
      [s1[0, 0] + s1[0, 1] + g1a[...], s1[1, 0] + s1[1, 1] + g1b[...]],
      axis=1)
  h1r = jnp.maximum(acc * dinv[:, None] + b1_p[...], 0.0)
  h2a = jnp.dot(h1r, w2t[...], preferred_element_type=jnp.float32)
  g2_out[...] = (h2a + c[...]) * dinv[:, None]


def _tc3_body(s2, g2, degp, b2_p, out):
  deg = degp[0, :, 0] + degp[1, :, 0] + 1.0
  dinv = lax.rsqrt(deg)
  out[...] = (s2[0, 0] + s2[0, 1] + g2[...]) * dinv[:, None] + b2_p[...]


def _row_spec(d):
  return pl.BlockSpec((R, d), lambda i: (i, 0))


def _part_spec(d):
  return pl.BlockSpec((NC, R, d), lambda i: (0, i, 0))


def _full_spec(shape):
  nd = len(shape)
  return pl.BlockSpec(shape, lambda i: (0,) * nd)


def kernel(graph, edge_index, rates, W1, b1, W2, b2, We1, be1, We2, be2):
  f32 = jnp.float32
  src = edge_index[0]
  dst = edge_index[1]
  pad = jnp.full((EPAD - E,), N, jnp.int32)
  src3 = jnp.concatenate([src, pad]).reshape(NW, CHUNKS, K)
  dst3 = jnp.concatenate([dst, pad]).reshape(NW, CHUNKS, K)
  edges4 = jnp.stack([src3, dst3], axis=2)
  graph_p = jnp.zeros((NPAD, FIN), f32).at[:N].set(graph)

  ones16 = jnp.ones((K, 16), f32)
  zeros16 = jnp.zeros((RPT, 16), f32)
  zeros64 = jnp.zeros((K, NCLS), f32)

  rates_p = jnp.zeros((1, 128), f32).at[0, :16].set(rates)
  we1_p = jnp.zeros((128, 128), f32).at[:16, :64].set(We1)
  be1_p = jnp.zeros((1, 128), f32).at[0, :64].set(be1)
  we2_p = jnp.zeros((128, 128), f32).at[:64, :].set(We2)
  be2_p = be2.reshape(1, HID)
  w2t = W2[:HID]
  w2b = W2[HID:]
  b1_p = b1.reshape(1, HID)
  b2_p = b2.reshape(1, NCLS)

  degp = _sc_deg(dst3, ones16, zeros16)

  grid = NPAD // R
  g1a, g1b, c = pl.pallas_call(
      _tc1_body,
      grid=(grid,),
      in_specs=[
          _row_spec(FIN), _full_spec((FIN, HID)), _part_spec(16),
          _full_spec((1, 128)), _full_spec((128, 128)), _full_spec((1, 128)),
          _full_spec((128, 128)), _full_spec((1, HID)),
          _full_spec((HID, NCLS)),
      ],
      out_specs=[_row_spec(NCLS), _row_spec(NCLS), _full_spec((1, NCLS))],
      out_shape=[
          jax.ShapeDtypeStruct((NPAD, NCLS), f32),
          jax.ShapeDtypeStruct((NPAD, NCLS), f32),
          jax.ShapeDtypeStruct((1, NCLS), f32),
      ],
  )(graph_p, W1, degp, rates_p, we1_p, be1_p, we2_p, be2_p, w2b)

  s1 = _sc_scatter_l1(g1a, g1b, edges4, zeros64)

  g2 = pl.pallas_call(
      _tc2_body,
      grid=(grid,),
      in_specs=[
          _row_spec(NCLS), _row_spec(NCLS), _part_spec(16),
          pl.BlockSpec((2, NC, R, NCLS), lambda i: (0, 0, i, 0)),
          _full_spec((1, HID)), _full_spec((HID, NCLS)),
          _full_spec((1, NCLS)),
      ],
      out_specs=_row_spec(NCLS),
      out_shape=jax.ShapeDtypeStruct((NPAD, NCLS), f32),
  )(g1a, g1b, degp, s1, b1_p, w2t, c)

  s2 = _sc_scatter_l2(g2, edges4, zeros64)

  out = pl.pallas_call(
      _tc3_body,
      grid=(grid,),
      in_specs=[
          pl.BlockSpec((1, NC, R, NCLS), lambda i: (0, 0, i, 0)),
          _row_spec(NCLS), _part_spec(16), _full_spec((1, NCLS)),
      ],
      out_specs=_row_spec(NCLS),
      out_shape=jax.ShapeDtypeStruct((NPAD, NCLS), f32),
  )(s2, g2, degp, b2_p)

  return out[:N]


# speedup vs baseline: 1.1889x; 1.1889x over previous
import functools

import jax
import jax.numpy as jnp
from jax import lax
from jax.experimental import pallas as pl
from jax.experimental.pallas import tpu as pltpu
from jax.experimental.pallas import tpu_sc as plsc

N = 10000
E = 320000
FIN = 128
HID = 128
NCLS = 64

NC = 2
NS = 16
NW = NC * NS

NPAD = 10240
K = 128
NCH = E // K
CHUNKS = NCH // NW
NEXTRA = NCH - CHUNKS * NW
RPT = NPAD // NS

_MESH = plsc.VectorSubcoreMesh(
    core_axis_name="c", subcore_axis_name="s", num_cores=NC, num_subcores=NS)

_SC_PARAMS = pltpu.CompilerParams(use_tc_tiling_on_sc=False)


def _worker_id():
  return lax.axis_index("c") * NS + lax.axis_index("s")


@functools.partial(
    pl.kernel,
    out_type=jax.ShapeDtypeStruct((NC, NPAD, 16), jnp.float32),
    mesh=_MESH,
    compiler_params=_SC_PARAMS,
    scratch_types=[
        pltpu.VMEM((CHUNKS + 1, K), jnp.int32),
        pltpu.VMEM((K, 16), jnp.float32),
        pltpu.VMEM((RPT, 16), jnp.float32),
        pltpu.VMEM_SHARED((NPAD, 16), jnp.float32),
    ],
)
def _sc_deg(ei2, ones16, zeros16, out, dstv, onesv, zbuf, accum):
  cid = lax.axis_index("c")
  sid = lax.axis_index("s")
  wid = _worker_id()
  pltpu.sync_copy(ei2.at[1, pl.ds(wid * CHUNKS, CHUNKS)],
                  dstv.at[pl.ds(0, CHUNKS)])
  nch = jnp.where(wid < NEXTRA, CHUNKS + 1, CHUNKS)

  @pl.when(wid < NEXTRA)
  def _():
    pltpu.sync_copy(ei2.at[1, pl.ds(NW * CHUNKS + wid, 1)],
                    dstv.at[pl.ds(CHUNKS, 1)])

  pltpu.sync_copy(ones16, onesv)
  pltpu.sync_copy(zeros16, zbuf)
  pltpu.sync_copy(zbuf, accum.at[pl.ds(sid * RPT, RPT)])
  plsc.subcore_barrier()

  def body(j, carry):
    pltpu.sync_copy(onesv, accum.at[dstv.at[j]], add=True)
    return carry

  lax.fori_loop(0, nch, body, 0)
  plsc.subcore_barrier()
  pltpu.sync_copy(accum.at[pl.ds(sid * RPT, RPT)], zbuf)
  pltpu.sync_copy(zbuf, out.at[cid, pl.ds(sid * RPT, RPT)])


NBUF = 3
NIDX = 2 * NBUF
D = NCLS


def _make_sc_scatter(nphases):
  scratch = [
      [pltpu.VMEM((2, K), jnp.int32)] * NIDX,
      [pltpu.VMEM((K, D), jnp.float32)] * NBUF,
      pltpu.VMEM((2, K), jnp.int32),
      pltpu.VMEM_SHARED((NPAD, D), jnp.float32),
      pltpu.VMEM_SHARED((NPAD, D), jnp.float32),
      [pltpu.SemaphoreType.DMA] * NIDX,
      [pltpu.SemaphoreType.DMA] * NBUF,
      [pltpu.SemaphoreType.DMA] * NBUF,
  ]

  @functools.partial(
      pl.kernel,
      out_type=jax.ShapeDtypeStruct((NC, NPAD, 128), jnp.float32),
      mesh=_MESH,
      compiler_params=_SC_PARAMS,
      scratch_types=scratch,
  )
  def sc_scatter(table, ei2, zrows, out, ibuf, rows, tidx, accum, g,
                 isem, gsem, ssem):
    cid = lax.axis_index("c")
    sid = lax.axis_index("s")
    wid = _worker_id()
    c0 = wid * CHUNKS
    ct = NW * CHUNKS + wid
    nzero = RPT // K

    @pl.when(wid < NEXTRA)
    def _():
      pltpu.sync_copy(ei2.at[0, pl.ds(ct, 1)], tidx.at[pl.ds(0, 1)])
      pltpu.sync_copy(ei2.at[1, pl.ds(ct, 1)], tidx.at[pl.ds(1, 1)])

    for p in range(nphases):
      col = pl.ds(p * D, D)
      for q in range(nzero):
        r0 = sid * RPT + q * K
        pltpu.sync_copy(table.at[pl.ds(r0, K), col], rows[0])
        pltpu.sync_copy(rows[0], g.at[pl.ds(r0, K)])
      pltpu.sync_copy(zrows, rows[0])
      for q in range(nzero):
        pltpu.sync_copy(rows[0], accum.at[pl.ds(sid * RPT + q * K, K)])
      plsc.subcore_barrier()

      for u in range(NIDX):
        pltpu.async_copy(ei2.at[0, pl.ds(c0 + u, 1)], ibuf[u].at[pl.ds(0, 1)],
                         isem[u])
        pltpu.async_copy(ei2.at[1, pl.ds(c0 + u, 1)], ibuf[u].at[pl.ds(1, 1)],
                         isem[u])
      for b in range(NBUF):
        pltpu.make_async_copy(ei2.at[0, pl.ds(c0, 2)], ibuf[b],
                              isem[b]).wait()
        pltpu.async_copy(g.at[ibuf[b].at[0]], rows[b], gsem[b])

      def group(gi, carry):
        base = gi * NIDX
        for u in range(NIDX):
          j = base + u
          b = u % NBUF
          pltpu.make_async_copy(zrows, rows[b], gsem[b]).wait()
          pltpu.async_copy(rows[b], accum.at[ibuf[u].at[1]], ssem[b],
                           add=True)
          pltpu.make_async_copy(zrows, rows[b], ssem[b]).wait()
          nj = j + NBUF
          nslot = (u + NBUF) % NIDX

          @pl.when(nj < CHUNKS)
          def _():
            pltpu.make_async_copy(ei2.at[0, pl.ds(c0, 2)], ibuf[nslot],
                                  isem[nslot]).wait()
            pltpu.async_copy(g.at[ibuf[nslot].at[0]], rows[b], gsem[b])

          pj = j + NIDX

          @pl.when(pj < CHUNKS)
          def _():
            pltpu.async_copy(ei2.at[0, pl.ds(c0 + pj, 1)],
                             ibuf[u].at[pl.ds(0, 1)], isem[u])
            pltpu.async_copy(ei2.at[1, pl.ds(c0 + pj, 1)],
                             ibuf[u].at[pl.ds(1, 1)], isem[u])

        return carry

      lax.fori_loop(0, CHUNKS // NIDX, group, 0)

      @pl.when(wid < NEXTRA)
      def _():
        pltpu.async_copy(g.at[tidx.at[0]], rows[0], gsem[0])
        pltpu.make_async_copy(zrows, rows[0], gsem[0]).wait()
        pltpu.sync_copy(rows[0], accum.at[tidx.at[1]], add=True)

      plsc.subcore_barrier()
      for q in range(nzero):
        r0 = sid * RPT + q * K
        pltpu.sync_copy(accum.at[pl.ds(r0, K)], rows[0])
        pltpu.sync_copy(rows[0], out.at[cid, pl.ds(r0, K), col])

  return sc_scatter


_sc_scatter_l1 = _make_sc_scatter(nphases=2)
_sc_scatter_l2 = _make_sc_scatter(nphases=1)


R = 1024


def _tc0_body(x, w1, rates_p, we1_p, be1_p, we2_p, be2_p, w2b, h1_out, c_out):
  h1_out[...] = jnp.dot(x[...], w1[...], preferred_element_type=jnp.float32)
  he = jnp.maximum(
      jnp.dot(rates_p[...], we1_p[...], preferred_element_type=jnp.float32)
      + be1_p[...], 0.0)
  rep = jnp.dot(he, we2_p[...], preferred_element_type=jnp.float32) + be2_p[...]
  c_out[...] = jnp.dot(rep, w2b[...], preferred_element_type=jnp.float32)


def _tc1_body(h1, degp, g1_out):
  deg = degp[0, :, 0] + degp[1, :, 0] + 1.0
  dinv = lax.rsqrt(deg)
  g1_out[...] = h1[...] * dinv[:, None]


def _tc2_body(g1, degp, s1, b1_p, w2t, c, g2_out):
  deg = degp[0, :, 0] + degp[1, :, 0] + 1.0
  dinv = lax.rsqrt(deg)
  acc = s1[0] + s1[1] + g1[...]
  h1r = jnp.maximum(acc * dinv[:, None] + b1_p[...], 0.0)
  h2a = jnp.dot(h1r, w2t[...], preferred_element_type=jnp.float32)
  g2 = (h2a + c[...]) * dinv[:, None]
  g2_out[...] = jnp.concatenate([g2, jnp.zeros_like(g2)], axis=1)


def _tc3_body(s2, g2, degp, b2_p, out):
  deg = degp[0, :, 0] + degp[1, :, 0] + 1.0
  dinv = lax.rsqrt(deg)
  acc = s2[0, :, :NCLS] + s2[1, :, :NCLS] + g2[:, :NCLS]
  out[...] = acc * dinv[:, None] + b2_p[...]


def _row_spec(d):
  return pl.BlockSpec((R, d), lambda i: (i, 0))


def _part_spec(d):
  return pl.BlockSpec((NC, R, d), lambda i: (0, i, 0))


def _full_spec(shape):
  nd = len(shape)
  return pl.BlockSpec(shape, lambda i: (0,) * nd)


def kernel(graph, edge_index, rates, W1, b1, W2, b2, We1, be1, We2, be2):
  f32 = jnp.float32
  graph_p = jnp.zeros((NPAD, FIN), f32).at[:N].set(graph)
  ei2 = edge_index.reshape(2, NCH, K)

  ones16 = jnp.ones((K, 16), f32)
  zeros16 = jnp.zeros((RPT, 16), f32)
  zeros64 = jnp.zeros((K, NCLS), f32)

  rates_p = jnp.zeros((1, 128), f32).at[0, :16].set(rates)
  we1_p = jnp.zeros((128, 128), f32).at[:16, :64].set(We1)
  be1_p = jnp.zeros((1, 128), f32).at[0, :64].set(be1)
  we2_p = jnp.zeros((128, 128), f32).at[:64, :].set(We2)
  be2_p = be2.reshape(1, HID)
  w2t = W2[:HID]
  w2b = W2[HID:]
  b1_p = b1.reshape(1, HID)
  b2_p = b2.reshape(1, NCLS)

  degp = _sc_deg(ei2, ones16, zeros16)

  grid = NPAD // R
  h1, c = pl.pallas_call(
      _tc0_body,
      grid=(grid,),
      in_specs=[
          _row_spec(FIN), _full_spec((FIN, HID)),
          _full_spec((1, 128)), _full_spec((128, 128)), _full_spec((1, 128)),
          _full_spec((128, 128)), _full_spec((1, HID)),
          _full_spec((HID, NCLS)),
      ],
      out_specs=[_row_spec(HID), _full_spec((1, NCLS))],
      out_shape=[
          jax.ShapeDtypeStruct((NPAD, HID), f32),
          jax.ShapeDtypeStruct((1, NCLS), f32),
      ],
  )(graph_p, W1, rates_p, we1_p, be1_p, we2_p, be2_p, w2b)

  g1 = pl.pallas_call(
      _tc1_body,
      grid=(grid,),
      in_specs=[_row_spec(HID), _part_spec(16)],
      out_specs=_row_spec(HID),
      out_shape=jax.ShapeDtypeStruct((NPAD, HID), f32),
  )(h1, degp)

  s1 = _sc_scatter_l1(g1, ei2, zeros64)

  g2 = pl.pallas_call(
      _tc2_body,
      grid=(grid,),
      in_specs=[
          _row_spec(HID), _part_spec(16), _part_spec(HID),
          _full_spec((1, HID)), _full_spec((HID, NCLS)),
          _full_spec((1, NCLS)),
      ],
      out_specs=_row_spec(HID),
      out_shape=jax.ShapeDtypeStruct((NPAD, HID), f32),
  )(g1, degp, s1, b1_p, w2t, c)

  s2 = _sc_scatter_l2(g2, ei2, zeros64)

  out = pl.pallas_call(
      _tc3_body,
      grid=(grid,),
      in_specs=[
          _part_spec(HID), _row_spec(HID), _part_spec(16),
          _full_spec((1, NCLS)),
      ],
      out_specs=_row_spec(NCLS),
      out_shape=jax.ShapeDtypeStruct((NPAD, NCLS), f32),
  )(s2, g2, degp, b2_p)

  return out[:N]


# speedup vs baseline: 1.2321x; 1.0363x over previous
import functools

import jax
import jax.numpy as jnp
from jax import lax
from jax.experimental import pallas as pl
from jax.experimental.pallas import tpu as pltpu
from jax.experimental.pallas import tpu_sc as plsc

N = 10000
E = 320000
FIN = 128
HID = 128
NCLS = 64

NC = 2
NS = 16
NW = NC * NS

NPAD = 10240
K = 128
NCH = E // K
CHUNKS = NCH // NW
NEXTRA = NCH - CHUNKS * NW
RPT = NPAD // NS

_MESH = plsc.VectorSubcoreMesh(
    core_axis_name="c", subcore_axis_name="s", num_cores=NC, num_subcores=NS)

_SC_PARAMS = pltpu.CompilerParams(use_tc_tiling_on_sc=False)


def _worker_id():
  return lax.axis_index("c") * NS + lax.axis_index("s")


@functools.partial(
    pl.kernel,
    out_type=jax.ShapeDtypeStruct((NC, NPAD, 16), jnp.float32),
    mesh=_MESH,
    compiler_params=_SC_PARAMS,
    scratch_types=[
        pltpu.VMEM((CHUNKS + 1, K), jnp.int32),
        pltpu.VMEM((K, 16), jnp.float32),
        pltpu.VMEM((RPT, 16), jnp.float32),
        pltpu.VMEM_SHARED((NPAD, 16), jnp.float32),
    ],
)
def _sc_deg(ei2, ones16, zeros16, out, dstv, onesv, zbuf, accum):
  cid = lax.axis_index("c")
  sid = lax.axis_index("s")
  wid = _worker_id()
  pltpu.sync_copy(ei2.at[1, pl.ds(wid * CHUNKS, CHUNKS)],
                  dstv.at[pl.ds(0, CHUNKS)])
  nch = jnp.where(wid < NEXTRA, CHUNKS + 1, CHUNKS)

  @pl.when(wid < NEXTRA)
  def _():
    pltpu.sync_copy(ei2.at[1, pl.ds(NW * CHUNKS + wid, 1)],
                    dstv.at[pl.ds(CHUNKS, 1)])

  pltpu.sync_copy(ones16, onesv)
  pltpu.sync_copy(zeros16, zbuf)
  pltpu.sync_copy(zbuf, accum.at[pl.ds(sid * RPT, RPT)])
  plsc.subcore_barrier()

  def body(j, carry):
    pltpu.sync_copy(onesv, accum.at[dstv.at[j]], add=True)
    return carry

  lax.fori_loop(0, nch, body, 0)
  plsc.subcore_barrier()
  pltpu.sync_copy(accum.at[pl.ds(sid * RPT, RPT)], zbuf)
  pltpu.sync_copy(zbuf, out.at[cid, pl.ds(sid * RPT, RPT)])


NBUF = 4
NIDX = 6
D = NCLS


def _make_sc_scatter(nphases):
  scratch = [
      [pltpu.VMEM((2, K), jnp.int32)] * NIDX,
      [pltpu.VMEM((K, D), jnp.float32)] * NBUF,
      pltpu.VMEM((2, K), jnp.int32),
      pltpu.VMEM_SHARED((NPAD, D), jnp.float32),
      pltpu.VMEM_SHARED((NPAD, D), jnp.float32),
      [pltpu.SemaphoreType.DMA] * NIDX,
      [pltpu.SemaphoreType.DMA] * NBUF,
      [pltpu.SemaphoreType.DMA] * NBUF,
  ]

  @functools.partial(
      pl.kernel,
      out_type=jax.ShapeDtypeStruct((NC, NPAD, 128), jnp.float32),
      mesh=_MESH,
      compiler_params=_SC_PARAMS,
      scratch_types=scratch,
  )
  def sc_scatter(table, ei2, zrows, out, ibuf, rows, tidx, accum, g,
                 isem, gsem, ssem):
    cid = lax.axis_index("c")
    sid = lax.axis_index("s")
    wid = _worker_id()
    c0 = wid * CHUNKS
    ct = NW * CHUNKS + wid
    nzero = RPT // K

    @pl.when(wid < NEXTRA)
    def _():
      pltpu.sync_copy(ei2.at[0, pl.ds(ct, 1)], tidx.at[pl.ds(0, 1)])
      pltpu.sync_copy(ei2.at[1, pl.ds(ct, 1)], tidx.at[pl.ds(1, 1)])

    for p in range(nphases):
      col = pl.ds(p * D, D)
      stripe = pl.ds(sid * RPT, RPT)
      pltpu.sync_copy(table.at[stripe, col], g.at[stripe])
      pltpu.sync_copy(zrows, rows[0])
      for q in range(nzero):
        pltpu.sync_copy(rows[0], accum.at[pl.ds(sid * RPT + q * K, K)])
      plsc.subcore_barrier()

      for u in range(NIDX):
        pltpu.async_copy(ei2.at[0, pl.ds(c0 + u, 1)], ibuf[u].at[pl.ds(0, 1)],
                         isem[u])
        pltpu.async_copy(ei2.at[1, pl.ds(c0 + u, 1)], ibuf[u].at[pl.ds(1, 1)],
                         isem[u])
      for b in range(NBUF):
        pltpu.make_async_copy(ei2.at[0, pl.ds(c0, 2)], ibuf[b],
                              isem[b]).wait()
        pltpu.async_copy(g.at[ibuf[b].at[0]], rows[b], gsem[b])

      def group(gi, carry):
        base = gi * NIDX
        for u in range(NIDX):
          j = base + u
          b = u % NBUF
          pltpu.make_async_copy(zrows, rows[b], gsem[b]).wait()
          pltpu.async_copy(rows[b], accum.at[ibuf[u].at[1]], ssem[b],
                           add=True)
          pltpu.make_async_copy(zrows, rows[b], ssem[b]).wait()
          nj = j + NBUF
          nslot = (u + NBUF) % NIDX

          @pl.when(nj < CHUNKS)
          def _():
            pltpu.make_async_copy(ei2.at[0, pl.ds(c0, 2)], ibuf[nslot],
                                  isem[nslot]).wait()
            pltpu.async_copy(g.at[ibuf[nslot].at[0]], rows[b], gsem[b])

          pj = j + NIDX

          @pl.when(pj < CHUNKS)
          def _():
            pltpu.async_copy(ei2.at[0, pl.ds(c0 + pj, 1)],
                             ibuf[u].at[pl.ds(0, 1)], isem[u])
            pltpu.async_copy(ei2.at[1, pl.ds(c0 + pj, 1)],
                             ibuf[u].at[pl.ds(1, 1)], isem[u])

        return carry

      lax.fori_loop(0, CHUNKS // NIDX, group, 0)

      @pl.when(wid < NEXTRA)
      def _():
        pltpu.async_copy(g.at[tidx.at[0]], rows[0], gsem[0])
        pltpu.make_async_copy(zrows, rows[0], gsem[0]).wait()
        pltpu.sync_copy(rows[0], accum.at[tidx.at[1]], add=True)

      plsc.subcore_barrier()
      stripe = pl.ds(sid * RPT, RPT)
      pltpu.sync_copy(accum.at[stripe], out.at[cid, stripe, col])

  return sc_scatter


_sc_scatter_l1 = _make_sc_scatter(nphases=2)
_sc_scatter_l2 = _make_sc_scatter(nphases=1)


R = 1024


def _tc0_body(x, w1, rates_p, we1_p, be1_p, we2_p, be2_p, w2b, h1_out, c_out):
  h1_out[...] = jnp.dot(x[...], w1[...], preferred_element_type=jnp.float32)
  he = jnp.maximum(
      jnp.dot(rates_p[...], we1_p[...], preferred_element_type=jnp.float32)
      + be1_p[...], 0.0)
  rep = jnp.dot(he, we2_p[...], preferred_element_type=jnp.float32) + be2_p[...]
  c_out[...] = jnp.dot(rep, w2b[...], preferred_element_type=jnp.float32)


def _tc1_body(h1, degp, g1_out):
  deg = degp[0, :, 0] + degp[1, :, 0] + 1.0
  dinv = lax.rsqrt(deg)
  g1_out[...] = h1[...] * dinv[:, None]


def _tc2_body(g1, degp, s1, b1_p, w2t, c, g2_out):
  deg = degp[0, :, 0] + degp[1, :, 0] + 1.0
  dinv = lax.rsqrt(deg)
  acc = s1[0] + s1[1] + g1[...]
  h1r = jnp.maximum(acc * dinv[:, None] + b1_p[...], 0.0)
  h2a = jnp.dot(h1r, w2t[...], preferred_element_type=jnp.float32)
  g2 = (h2a + c[...]) * dinv[:, None]
  g2_out[...] = jnp.concatenate([g2, jnp.zeros_like(g2)], axis=1)


def _tc3_body(s2, g2, degp, b2_p, out):
  deg = degp[0, :, 0] + degp[1, :, 0] + 1.0
  dinv = lax.rsqrt(deg)
  acc = s2[0, :, :NCLS] + s2[1, :, :NCLS] + g2[:, :NCLS]
  out[...] = acc * dinv[:, None] + b2_p[...]


def _row_spec(d):
  return pl.BlockSpec((R, d), lambda i: (i, 0))


def _part_spec(d):
  return pl.BlockSpec((NC, R, d), lambda i: (0, i, 0))


def _full_spec(shape):
  nd = len(shape)
  return pl.BlockSpec(shape, lambda i: (0,) * nd)


def kernel(graph, edge_index, rates, W1, b1, W2, b2, We1, be1, We2, be2):
  f32 = jnp.float32
  graph_p = jnp.zeros((NPAD, FIN), f32).at[:N].set(graph)
  ei2 = edge_index.reshape(2, NCH, K)

  ones16 = jnp.ones((K, 16), f32)
  zeros16 = jnp.zeros((RPT, 16), f32)
  zeros64 = jnp.zeros((K, NCLS), f32)

  rates_p = jnp.zeros((1, 128), f32).at[0, :16].set(rates)
  we1_p = jnp.zeros((128, 128), f32).at[:16, :64].set(We1)
  be1_p = jnp.zeros((1, 128), f32).at[0, :64].set(be1)
  we2_p = jnp.zeros((128, 128), f32).at[:64, :].set(We2)
  be2_p = be2.reshape(1, HID)
  w2t = W2[:HID]
  w2b = W2[HID:]
  b1_p = b1.reshape(1, HID)
  b2_p = b2.reshape(1, NCLS)

  degp = _sc_deg(ei2, ones16, zeros16)

  grid = NPAD // R
  h1, c = pl.pallas_call(
      _tc0_body,
      grid=(grid,),
      in_specs=[
          _row_spec(FIN), _full_spec((FIN, HID)),
          _full_spec((1, 128)), _full_spec((128, 128)), _full_spec((1, 128)),
          _full_spec((128, 128)), _full_spec((1, HID)),
          _full_spec((HID, NCLS)),
      ],
      out_specs=[_row_spec(HID), _full_spec((1, NCLS))],
      out_shape=[
          jax.ShapeDtypeStruct((NPAD, HID), f32),
          jax.ShapeDtypeStruct((1, NCLS), f32),
      ],
  )(graph_p, W1, rates_p, we1_p, be1_p, we2_p, be2_p, w2b)

  g1 = pl.pallas_call(
      _tc1_body,
      grid=(grid,),
      in_specs=[_row_spec(HID), _part_spec(16)],
      out_specs=_row_spec(HID),
      out_shape=jax.ShapeDtypeStruct((NPAD, HID), f32),
  )(h1, degp)

  s1 = _sc_scatter_l1(g1, ei2, zeros64)

  g2 = pl.pallas_call(
      _tc2_body,
      grid=(grid,),
      in_specs=[
          _row_spec(HID), _part_spec(16), _part_spec(HID),
          _full_spec((1, HID)), _full_spec((HID, NCLS)),
          _full_spec((1, NCLS)),
      ],
      out_specs=_row_spec(HID),
      out_shape=jax.ShapeDtypeStruct((NPAD, HID), f32),
  )(g1, degp, s1, b1_p, w2t, c)

  s2 = _sc_scatter_l2(g2, ei2, zeros64)

  out = pl.pallas_call(
      _tc3_body,
      grid=(grid,),
      in_specs=[
          _part_spec(HID), _row_spec(HID), _part_spec(16),
          _full_spec((1, NCLS)),
      ],
      out_specs=_row_spec(NCLS),
      out_shape=jax.ShapeDtypeStruct((NPAD, NCLS), f32),
  )(s2, g2, degp, b2_p)

  return out[:N]


# speedup vs baseline: 1.2372x; 1.0042x over previous
import functools

import jax
import jax.numpy as jnp
from jax import lax
from jax.experimental import pallas as pl
from jax.experimental.pallas import tpu as pltpu
from jax.experimental.pallas import tpu_sc as plsc

N = 10000
E = 320000
FIN = 128
HID = 128
NCLS = 64

NC = 2
NS = 16
NW = NC * NS

NPAD = 10240
K = 128
NCH = E // K
CHUNKS = NCH // NW
NEXTRA = NCH - CHUNKS * NW
RPT = NPAD // NS

_MESH = plsc.VectorSubcoreMesh(
    core_axis_name="c", subcore_axis_name="s", num_cores=NC, num_subcores=NS)

_SC_PARAMS = pltpu.CompilerParams(use_tc_tiling_on_sc=False)


def _worker_id():
  return lax.axis_index("c") * NS + lax.axis_index("s")


@functools.partial(
    pl.kernel,
    out_type=jax.ShapeDtypeStruct((NC, NPAD, 16), jnp.float32),
    mesh=_MESH,
    compiler_params=_SC_PARAMS,
    scratch_types=[
        pltpu.VMEM((CHUNKS + 1, K), jnp.int32),
        pltpu.VMEM((K, 16), jnp.float32),
        pltpu.VMEM((RPT, 16), jnp.float32),
        pltpu.VMEM_SHARED((NPAD, 16), jnp.float32),
    ],
)
def _sc_deg(ei2, ones16, zeros16, out, dstv, onesv, zbuf, accum):
  cid = lax.axis_index("c")
  sid = lax.axis_index("s")
  wid = _worker_id()
  pltpu.sync_copy(ei2.at[1, pl.ds(wid * CHUNKS, CHUNKS)],
                  dstv.at[pl.ds(0, CHUNKS)])
  nch = jnp.where(wid < NEXTRA, CHUNKS + 1, CHUNKS)

  @pl.when(wid < NEXTRA)
  def _():
    pltpu.sync_copy(ei2.at[1, pl.ds(NW * CHUNKS + wid, 1)],
                    dstv.at[pl.ds(CHUNKS, 1)])

  pltpu.sync_copy(ones16, onesv)
  pltpu.sync_copy(zeros16, zbuf)
  pltpu.sync_copy(zbuf, accum.at[pl.ds(sid * RPT, RPT)])
  plsc.subcore_barrier()

  def body(j, carry):
    pltpu.sync_copy(onesv, accum.at[dstv.at[j]], add=True)
    return carry

  lax.fori_loop(0, nch, body, 0)
  plsc.subcore_barrier()
  pltpu.sync_copy(accum.at[pl.ds(sid * RPT, RPT)], zbuf)
  pltpu.sync_copy(zbuf, out.at[cid, pl.ds(sid * RPT, RPT)])


NBUF = 3
NIDX = 6
D = NCLS


def _make_sc_scatter(nphases):
  scratch = [
      [pltpu.VMEM((2, K), jnp.int32)] * NIDX,
      [pltpu.VMEM((K, D), jnp.float32)] * NBUF,
      pltpu.VMEM((2, K), jnp.int32),
      pltpu.VMEM_SHARED((NPAD, D), jnp.float32),
      pltpu.VMEM_SHARED((NPAD, D), jnp.float32),
      [pltpu.SemaphoreType.DMA] * NIDX,
      [pltpu.SemaphoreType.DMA] * NBUF,
      [pltpu.SemaphoreType.DMA] * NBUF,
  ]

  @functools.partial(
      pl.kernel,
      out_type=jax.ShapeDtypeStruct((NC, NPAD, 128), jnp.float32),
      mesh=_MESH,
      compiler_params=_SC_PARAMS,
      scratch_types=scratch,
  )
  def sc_scatter(table, ei2, zrows, out, ibuf, rows, tidx, accum, g,
                 isem, gsem, ssem):
    cid = lax.axis_index("c")
    sid = lax.axis_index("s")
    wid = _worker_id()
    c0 = wid * CHUNKS
    ct = NW * CHUNKS + wid
    nzero = RPT // K

    @pl.when(wid < NEXTRA)
    def _():
      pltpu.sync_copy(ei2.at[0, pl.ds(ct, 1)], tidx.at[pl.ds(0, 1)])
      pltpu.sync_copy(ei2.at[1, pl.ds(ct, 1)], tidx.at[pl.ds(1, 1)])

    for p in range(nphases):
      col = pl.ds(p * D, D)
      stripe = pl.ds(sid * RPT, RPT)
      pltpu.sync_copy(table.at[stripe, col], g.at[stripe])
      pltpu.sync_copy(zrows, rows[0])
      for q in range(nzero):
        pltpu.sync_copy(rows[0], accum.at[pl.ds(sid * RPT + q * K, K)])
      plsc.subcore_barrier()

      for u in range(NIDX):
        pltpu.async_copy(ei2.at[0, pl.ds(c0 + u, 1)], ibuf[u].at[pl.ds(0, 1)],
                         isem[u])
        pltpu.async_copy(ei2.at[1, pl.ds(c0 + u, 1)], ibuf[u].at[pl.ds(1, 1)],
                         isem[u])
      for b in range(NBUF):
        pltpu.make_async_copy(ei2.at[0, pl.ds(c0, 2)], ibuf[b],
                              isem[b]).wait()
        pltpu.async_copy(g.at[ibuf[b].at[0]], rows[b], gsem[b])

      def group(gi, carry):
        base = gi * NIDX
        for u in range(NIDX):
          j = base + u
          b = u % NBUF
          pltpu.make_async_copy(zrows, rows[b], gsem[b]).wait()
          pltpu.async_copy(rows[b], accum.at[ibuf[u].at[1]], ssem[b],
                           add=True)
          pltpu.make_async_copy(zrows, rows[b], ssem[b]).wait()
          nj = j + NBUF
          nslot = (u + NBUF) % NIDX

          @pl.when(nj < CHUNKS)
          def _():
            pltpu.make_async_copy(ei2.at[0, pl.ds(c0, 2)], ibuf[nslot],
                                  isem[nslot]).wait()
            pltpu.async_copy(g.at[ibuf[nslot].at[0]], rows[b], gsem[b])

          pj = j + NIDX

          @pl.when(pj < CHUNKS)
          def _():
            pltpu.async_copy(ei2.at[0, pl.ds(c0 + pj, 1)],
                             ibuf[u].at[pl.ds(0, 1)], isem[u])
            pltpu.async_copy(ei2.at[1, pl.ds(c0 + pj, 1)],
                             ibuf[u].at[pl.ds(1, 1)], isem[u])

        return carry

      lax.fori_loop(0, CHUNKS // NIDX, group, 0)

      @pl.when(wid < NEXTRA)
      def _():
        pltpu.async_copy(g.at[tidx.at[0]], rows[0], gsem[0])
        pltpu.make_async_copy(zrows, rows[0], gsem[0]).wait()
        pltpu.sync_copy(rows[0], accum.at[tidx.at[1]], add=True)

      plsc.subcore_barrier()
      stripe = pl.ds(sid * RPT, RPT)
      pltpu.sync_copy(accum.at[stripe], out.at[cid, stripe, col])

  return sc_scatter


_sc_scatter_l1 = _make_sc_scatter(nphases=2)
_sc_scatter_l2 = _make_sc_scatter(nphases=1)


R = 1024


def _tc0_body(x, w1, rates_p, we1_p, be1_p, we2_p, be2_p, w2b, h1_out, c_out):
  h1_out[...] = jnp.dot(x[...], w1[...], preferred_element_type=jnp.float32)
  he = jnp.maximum(
      jnp.dot(rates_p[...], we1_p[...], preferred_element_type=jnp.float32)
      + be1_p[...], 0.0)
  rep = jnp.dot(he, we2_p[...], preferred_element_type=jnp.float32) + be2_p[...]
  c_out[...] = jnp.dot(rep, w2b[...], preferred_element_type=jnp.float32)


def _tc1_body(h1, degp, g1_out):
  deg = degp[0, :, 0] + degp[1, :, 0] + 1.0
  dinv = lax.rsqrt(deg)
  g1_out[...] = h1[...] * dinv[:, None]


def _tc2_body(g1, degp, s1, b1_p, w2t, c, g2_out):
  deg = degp[0, :, 0] + degp[1, :, 0] + 1.0
  dinv = lax.rsqrt(deg)
  acc = s1[0] + s1[1] + g1[...]
  h1r = jnp.maximum(acc * dinv[:, None] + b1_p[...], 0.0)
  h2a = jnp.dot(h1r, w2t[...], preferred_element_type=jnp.float32)
  g2 = (h2a + c[...]) * dinv[:, None]
  g2_out[...] = jnp.concatenate([g2, jnp.zeros_like(g2)], axis=1)


def _tc3_body(s2, g2, degp, b2_p, out):
  deg = degp[0, :, 0] + degp[1, :, 0] + 1.0
  dinv = lax.rsqrt(deg)
  acc = s2[0, :, :NCLS] + s2[1, :, :NCLS] + g2[:, :NCLS]
  out[...] = acc * dinv[:, None] + b2_p[...]


def _row_spec(d):
  return pl.BlockSpec((R, d), lambda i: (i, 0))


def _part_spec(d):
  return pl.BlockSpec((NC, R, d), lambda i: (0, i, 0))


def _full_spec(shape):
  nd = len(shape)
  return pl.BlockSpec(shape, lambda i: (0,) * nd)


def kernel(graph, edge_index, rates, W1, b1, W2, b2, We1, be1, We2, be2):
  f32 = jnp.float32
  graph_p = jnp.zeros((NPAD, FIN), f32).at[:N].set(graph)
  ei2 = edge_index.reshape(2, NCH, K)

  ones16 = jnp.ones((K, 16), f32)
  zeros16 = jnp.zeros((RPT, 16), f32)
  zeros64 = jnp.zeros((K, NCLS), f32)

  rates_p = jnp.zeros((1, 128), f32).at[0, :16].set(rates)
  we1_p = jnp.zeros((128, 128), f32).at[:16, :64].set(We1)
  be1_p = jnp.zeros((1, 128), f32).at[0, :64].set(be1)
  we2_p = jnp.zeros((128, 128), f32).at[:64, :].set(We2)
  be2_p = be2.reshape(1, HID)
  w2t = W2[:HID]
  w2b = W2[HID:]
  b1_p = b1.reshape(1, HID)
  b2_p = b2.reshape(1, NCLS)

  degp = _sc_deg(ei2, ones16, zeros16)

  grid = NPAD // R
  h1, c = pl.pallas_call(
      _tc0_body,
      grid=(grid,),
      in_specs=[
          _row_spec(FIN), _full_spec((FIN, HID)),
          _full_spec((1, 128)), _full_spec((128, 128)), _full_spec((1, 128)),
          _full_spec((128, 128)), _full_spec((1, HID)),
          _full_spec((HID, NCLS)),
      ],
      out_specs=[_row_spec(HID), _full_spec((1, NCLS))],
      out_shape=[
          jax.ShapeDtypeStruct((NPAD, HID), f32),
          jax.ShapeDtypeStruct((1, NCLS), f32),
      ],
  )(graph_p, W1, rates_p, we1_p, be1_p, we2_p, be2_p, w2b)

  g1 = pl.pallas_call(
      _tc1_body,
      grid=(grid,),
      in_specs=[_row_spec(HID), _part_spec(16)],
      out_specs=_row_spec(HID),
      out_shape=jax.ShapeDtypeStruct((NPAD, HID), f32),
  )(h1, degp)

  s1 = _sc_scatter_l1(g1, ei2, zeros64)

  g2 = pl.pallas_call(
      _tc2_body,
      grid=(grid,),
      in_specs=[
          _row_spec(HID), _part_spec(16), _part_spec(HID),
          _full_spec((1, HID)), _full_spec((HID, NCLS)),
          _full_spec((1, NCLS)),
      ],
      out_specs=_row_spec(HID),
      out_shape=jax.ShapeDtypeStruct((NPAD, HID), f32),
  )(g1, degp, s1, b1_p, w2t, c)

  s2 = _sc_scatter_l2(g2, ei2, zeros64)

  out = pl.pallas_call(
      _tc3_body,
      grid=(grid,),
      in_specs=[
          _part_spec(HID), _row_spec(HID), _part_spec(16),
          _full_spec((1, NCLS)),
      ],
      out_specs=_row_spec(NCLS),
      out_shape=jax.ShapeDtypeStruct((NPAD, NCLS), f32),
  )(s2, g2, degp, b2_p)

  return out[:N]
